# baseline (speedup 1.0000x reference)
import functools
import math

import jax
import jax.numpy as jnp
from jax import lax
from jax.experimental import pallas as pl
from jax.experimental.pallas import tpu as pltpu
from jax.experimental.pallas import tpu_sc as plsc

NC = 2
NS = 16
CHUNK = 128

BN_SCALE = 1.0 / math.sqrt(1.0 + 1e-5)


def _sc_mesh():
  return plsc.VectorSubcoreMesh(
      core_axis_name="c", subcore_axis_name="s", num_cores=NC,
      num_subcores=NS)


def _sc_degree(dst2, n):
  nchunks = dst2.shape[0]
  pc = nchunks // NC
  pt = pc // NS
  extra = pc - pt * NS
  rows_t = (n // NS) // 8 * 8
  rows_tail = n - NS * rows_t

  GS_D = 26
  WAVE = 13
  assert pt % GS_D == 0 and GS_D % WAVE == 0

  def body(dst_hbm, out0, out1, didx, xidx_v, ones_v, stage_v, acc_sh, sem):
    c = lax.axis_index("c")
    s = lax.axis_index("s")
    base = c * pc + s * pt
    xidx_ref = xidx_v.at[0, 0]

    def setones(i, _):
      ones_v[pl.ds(i * 16, 16)] = jnp.ones((16,), jnp.float32)
      stage_v[pl.ds(i * 16, 16)] = jnp.zeros((16,), jnp.float32)
      return 0
    lax.fori_loop(0, CHUNK // 16, setones, 0, unroll=True)

    def zinit(i, _):
      pltpu.sync_copy(stage_v,
                      acc_sh.at[pl.ds(s * rows_t + i * CHUNK, CHUNK)])
      return 0
    lax.fori_loop(0, rows_t // CHUNK, zinit, 0)
    rem = rows_t - (rows_t // CHUNK) * CHUNK
    if rem:
      pltpu.sync_copy(stage_v.at[pl.ds(0, rem)],
                      acc_sh.at[pl.ds(s * rows_t + rows_t - rem, rem)])

    @pl.when(s == 0)
    def _():
      pltpu.sync_copy(stage_v.at[pl.ds(0, rows_tail)],
                      acc_sh.at[pl.ds(NS * rows_t, rows_tail)])

    plsc.subcore_barrier()

    def group(gd, _):
      pltpu.sync_copy(dst_hbm.at[pl.ds(base + gd * GS_D, GS_D)], didx)

      def wave(wv, _):
        for b in range(WAVE):
          pltpu.async_copy(ones_v, acc_sh.at[didx.at[wv * WAVE + b, 0]],
                           sem, add=True)
        for b in range(WAVE):
          pltpu.make_async_copy(ones_v, acc_sh.at[didx.at[0, 0]], sem).wait()
        return 0
      lax.fori_loop(0, GS_D // WAVE, wave, 0)
      return 0
    lax.fori_loop(0, pt // GS_D, group, 0)

    @pl.when(s < extra)
    def _():
      pltpu.sync_copy(dst_hbm.at[pl.ds(c * pc + pt * NS + s, 1)], xidx_v)
      pltpu.sync_copy(ones_v, acc_sh.at[xidx_ref], add=True)

    plsc.subcore_barrier()

    def writeout(out_hbm):
      def wchunk(i, _):
        pltpu.sync_copy(acc_sh.at[pl.ds(s * rows_t + i * CHUNK, CHUNK)],
                        stage_v)
        pltpu.sync_copy(stage_v,
                        out_hbm.at[pl.ds(s * rows_t + i * CHUNK, CHUNK)])
        return 0
      lax.fori_loop(0, rows_t // CHUNK, wchunk, 0)
      if rem:
        pltpu.sync_copy(acc_sh.at[pl.ds(s * rows_t + rows_t - rem, rem)],
                        stage_v.at[pl.ds(0, rem)])
        pltpu.sync_copy(stage_v.at[pl.ds(0, rem)],
                        out_hbm.at[pl.ds(s * rows_t + rows_t - rem, rem)])

      @pl.when(s == 0)
      def _():
        pltpu.sync_copy(acc_sh.at[pl.ds(NS * rows_t, rows_tail)],
                        stage_v.at[pl.ds(0, rows_tail)])
        pltpu.sync_copy(stage_v.at[pl.ds(0, rows_tail)],
                        out_hbm.at[pl.ds(NS * rows_t, rows_tail)])

    @pl.when(c == 0)
    def _():
      writeout(out0)

    @pl.when(c == 1)
    def _():
      writeout(out1)

  f = pl.kernel(
      body,
      out_type=[jax.ShapeDtypeStruct((n,), jnp.float32),
                jax.ShapeDtypeStruct((n,), jnp.float32)],
      mesh=_sc_mesh(),
      scratch_types=[
          pltpu.VMEM((GS_D, 1, CHUNK), jnp.int32),
          pltpu.VMEM((1, 1, CHUNK), jnp.int32),
          pltpu.VMEM((CHUNK,), jnp.float32),
          pltpu.VMEM((CHUNK,), jnp.float32),
          pltpu.VMEM_SHARED((n,), jnp.float32),
          pltpu.SemaphoreType.DMA,
      ],
  )
  return f(dst2)


def _sc_edge_sum_part(p, src2, dst2):
  n, d = p.shape
  nchunks = src2.shape[0]
  pc = nchunks // NC
  pt = pc // NS
  extra = pc - pt * NS
  GS = 26
  assert pt % GS == 0
  rows_t = (n // NS) // 8 * 8
  rows_tail = n - NS * rows_t
  sub = [CHUNK] * (rows_t // CHUNK)
  if rows_t % CHUNK:
    sub.append(rows_t % CHUNK)

  def body(p_hbm, src_hbm, dst_hbm, out0, out1,
           sidx, didx, sidx2, didx2, xsidx, xdidx, rows_v, rows_w,
           acc_sh, sem_a, sem_b, sem_sa, sem_sb, sem_ia, sem_ib):
    c = lax.axis_index("c")
    s = lax.axis_index("s")
    base = c * pc + s * pt
    xsidx_ref = xsidx.at[0, 0]
    xdidx_ref = xdidx.at[0, 0]

    def zrow(i, _):
      row = rows_v.at[i]
      def zcol(j, _):
        row[pl.ds(j * 16, 16)] = jnp.zeros((16,), jnp.float32)
        return 0
      lax.fori_loop(0, d // 16, zcol, 0, unroll=True)
      return 0
    lax.fori_loop(0, CHUNK, zrow, 0)

    off = 0
    for sz in sub:
      pltpu.sync_copy(rows_v.at[pl.ds(0, sz)],
                      acc_sh.at[pl.ds(s * rows_t + off, sz)])
      off += sz

    @pl.when(s == 0)
    def _():
      pltpu.sync_copy(rows_v.at[pl.ds(0, rows_tail)],
                      acc_sh.at[pl.ds(NS * rows_t, rows_tail)])

    plsc.subcore_barrier()

    ngroups = pt // GS
    sbufs = [sidx, sidx2]
    dbufs = [didx, didx2]
    pltpu.async_copy(src_hbm.at[pl.ds(base, GS)], sidx, sem_ia)
    pltpu.async_copy(dst_hbm.at[pl.ds(base, GS)], didx, sem_ib)

    for gg in range(ngroups):
      si = sbufs[gg % 2]
      di = dbufs[gg % 2]
      pltpu.make_async_copy(src_hbm.at[pl.ds(base, GS)], si, sem_ia).wait()
      pltpu.make_async_copy(dst_hbm.at[pl.ds(base, GS)], di, sem_ib).wait()
      if gg + 1 < ngroups:
        nb = base + (gg + 1) * GS
        pltpu.async_copy(src_hbm.at[pl.ds(nb, GS)], sbufs[(gg + 1) % 2],
                         sem_ia)
        pltpu.async_copy(dst_hbm.at[pl.ds(nb, GS)], dbufs[(gg + 1) % 2],
                         sem_ib)
      pltpu.async_copy(p_hbm.at[si.at[0, 0]], rows_v, sem_a)
      pltpu.async_copy(p_hbm.at[si.at[1, 0]], rows_w, sem_b)

      def pair(k, _):
        j0 = 2 * k
        pltpu.make_async_copy(p_hbm.at[si.at[0, 0]], rows_v, sem_a).wait()
        ca = pltpu.async_copy(rows_v, acc_sh.at[di.at[j0, 0]], sem_sa,
                              add=True)
        pltpu.make_async_copy(p_hbm.at[si.at[0, 0]], rows_w, sem_b).wait()
        cb = pltpu.async_copy(rows_w, acc_sh.at[di.at[j0 + 1, 0]], sem_sb,
                              add=True)
        ca.wait()

        @pl.when(j0 + 2 < GS)
        def _():
          pltpu.async_copy(p_hbm.at[si.at[j0 + 2, 0]], rows_v, sem_a)

        cb.wait()

        @pl.when(j0 + 3 < GS)
        def _():
          pltpu.async_copy(p_hbm.at[si.at[j0 + 3, 0]], rows_w, sem_b)

        return 0
      lax.fori_loop(0, GS // 2, pair, 0)

    @pl.when(s < extra)
    def _():
      pltpu.sync_copy(src_hbm.at[pl.ds(c * pc + pt * NS + s, 1)], xsidx)
      pltpu.sync_copy(dst_hbm.at[pl.ds(c * pc + pt * NS + s, 1)], xdidx)
      pltpu.async_copy(p_hbm.at[xsidx_ref], rows_v, sem_a).wait()
      pltpu.sync_copy(rows_v, acc_sh.at[xdidx_ref], add=True)

    plsc.subcore_barrier()

    def writeout(out_hbm):
      woff = 0
      for sz in sub:
        pltpu.sync_copy(acc_sh.at[pl.ds(s * rows_t + woff, sz)],
                        rows_v.at[pl.ds(0, sz)])
        pltpu.sync_copy(rows_v.at[pl.ds(0, sz)],
                        out_hbm.at[pl.ds(s * rows_t + woff, sz)])
        woff += sz

      @pl.when(s == 0)
      def _():
        pltpu.sync_copy(acc_sh.at[pl.ds(NS * rows_t, rows_tail)],
                        rows_v.at[pl.ds(0, rows_tail)])
        pltpu.sync_copy(rows_v.at[pl.ds(0, rows_tail)],
                        out_hbm.at[pl.ds(NS * rows_t, rows_tail)])

    @pl.when(c == 0)
    def _():
      writeout(out0)

    @pl.when(c == 1)
    def _():
      writeout(out1)

  f = pl.kernel(
      body,
      out_type=[jax.ShapeDtypeStruct((n, d), jnp.float32),
                jax.ShapeDtypeStruct((n, d), jnp.float32)],
      mesh=_sc_mesh(),
      scratch_types=[
          pltpu.VMEM((GS, 1, CHUNK), jnp.int32),
          pltpu.VMEM((GS, 1, CHUNK), jnp.int32),
          pltpu.VMEM((GS, 1, CHUNK), jnp.int32),
          pltpu.VMEM((GS, 1, CHUNK), jnp.int32),
          pltpu.VMEM((1, 1, CHUNK), jnp.int32),
          pltpu.VMEM((1, 1, CHUNK), jnp.int32),
          pltpu.VMEM((CHUNK, d), jnp.float32),
          pltpu.VMEM((CHUNK, d), jnp.float32),
          pltpu.VMEM_SHARED((n, d), jnp.float32),
          pltpu.SemaphoreType.DMA,
          pltpu.SemaphoreType.DMA,
          pltpu.SemaphoreType.DMA,
          pltpu.SemaphoreType.DMA,
          pltpu.SemaphoreType.DMA,
          pltpu.SemaphoreType.DMA,
      ],
  )
  return f(p, src2, dst2)


def _tc_prep(x, deg0, deg1):
  n, d = x.shape
  r = 2000
  grid = n // r

  def body(x_ref, d0_ref, d1_ref, di_ref, p_ref):
    deg = d0_ref[...] + d1_ref[...] + 1.0
    di = lax.rsqrt(deg)
    di_ref[...] = di
    p_ref[...] = x_ref[...] * di

  return pl.pallas_call(
      body,
      grid=(grid,),
      in_specs=[
          pl.BlockSpec((r, d), lambda i: (i, 0)),
          pl.BlockSpec((r, 1), lambda i: (i, 0)),
          pl.BlockSpec((r, 1), lambda i: (i, 0)),
      ],
      out_specs=[
          pl.BlockSpec((r, 1), lambda i: (i, 0)),
          pl.BlockSpec((r, d), lambda i: (i, 0)),
      ],
      out_shape=[
          jax.ShapeDtypeStruct((n, 1), jnp.float32),
          jax.ShapeDtypeStruct((n, d), jnp.float32),
      ],
  )(x, deg0.reshape(n, 1), deg1.reshape(n, 1))


def _tc_dense(a_parts, p_parts, dinv, w, b, g, be, split_in, out_mode):
  n = a_parts[0].shape[0]
  din = w.shape[0]
  dout = w.shape[1]
  r = 2000
  grid = n // r

  def body(*refs):
    a_refs = refs[:len(a_parts)]
    p_refs = refs[len(a_parts):len(a_parts) + len(p_parts)]
    di, w_ref, b_ref, g_ref, be_ref = refs[len(a_parts) + len(p_parts):-
                                           (2 if out_mode == 'half_p' else 1)]
    outs = refs[-(2 if out_mode == 'half_p' else 1):]
    dv = di[...]
    def up(r_):
      return r_[...].astype(jnp.float32)
    if split_in:
      t = jnp.concatenate(
          [up(a_refs[0]) + up(a_refs[1]) + up(p_refs[0]),
           up(a_refs[2]) + up(a_refs[3]) + up(p_refs[1])], axis=1) * dv
    else:
      t = (up(a_refs[0]) + up(a_refs[1]) + up(p_refs[0])) * dv
    h = jnp.dot(t, w_ref[...], preferred_element_type=jnp.float32,
                precision=lax.Precision.HIGHEST)
    h = (h + b_ref[...]) * (g_ref[...] * BN_SCALE) + be_ref[...]
    h = jnp.maximum(h, 0.0)
    if out_mode == 'h':
      outs[0][...] = h
    elif out_mode == 'full_p':
      outs[0][...] = h * dv
    else:
      pn = h * dv
      outs[0][...] = pn[:, :dout // 2]
      outs[1][...] = pn[:, dout // 2:]

  out_dt = jnp.float32
  if out_mode == 'half_p':
    out_specs = [pl.BlockSpec((r, dout // 2), lambda i: (i, 0)),
                 pl.BlockSpec((r, dout // 2), lambda i: (i, 0))]
    out_shape = [jax.ShapeDtypeStruct((n, dout // 2), out_dt),
                 jax.ShapeDtypeStruct((n, dout // 2), out_dt)]
  else:
    out_specs = [pl.BlockSpec((r, dout), lambda i: (i, 0))]
    out_shape = [jax.ShapeDtypeStruct((n, dout), out_dt)]

  in_specs = (
      [pl.BlockSpec((r, a.shape[1]), lambda i: (i, 0)) for a in a_parts] +
      [pl.BlockSpec((r, p.shape[1]), lambda i: (i, 0)) for p in p_parts] +
      [pl.BlockSpec((r, 1), lambda i: (i, 0)),
       pl.BlockSpec((din, dout), lambda i: (0, 0)),
       pl.BlockSpec((1, dout), lambda i: (0, 0)),
       pl.BlockSpec((1, dout), lambda i: (0, 0)),
       pl.BlockSpec((1, dout), lambda i: (0, 0))])

  res = pl.pallas_call(
      body,
      grid=(grid,),
      in_specs=in_specs,
      out_specs=out_specs,
      out_shape=out_shape,
  )(*a_parts, *p_parts, dinv,
    w, b.reshape(1, dout), g.reshape(1, dout), be.reshape(1, dout))
  return res


def _tc_pool_mlp(a_parts, p_parts, dinv, w, b, g, be, batch, ngraphs,
                 wm1, bm1, wm2, bm2, wm3, bm3):
  n = a_parts[0].shape[0]
  din = w.shape[0]
  d = w.shape[1]
  r = 2000
  grid = n // r
  d1 = wm1.shape[1]
  d2 = wm2.shape[1]
  d3 = wm3.shape[1]

  def gelu(z):
    return 0.5 * z * (1.0 + lax.erf(z * (1.0 / math.sqrt(2.0))))

  def body(al0, al1, ah0, ah1, plo, phi, di, w_ref, b_ref2, g_ref, be_ref,
           b_ref, w1, v1, w2, v2, w3, v3, out,
           sum_acc, max_acc, cnt_acc):
    i = pl.program_id(0)

    @pl.when(i == 0)
    def _():
      sum_acc[...] = jnp.zeros_like(sum_acc)
      cnt_acc[...] = jnp.zeros_like(cnt_acc)
      max_acc[...] = jnp.full_like(max_acc, -jnp.inf)

    dv = di[...]
    def up(r_):
      return r_[...].astype(jnp.float32)
    t = jnp.concatenate([up(al0) + up(al1) + up(plo),
                         up(ah0) + up(ah1) + up(phi)], axis=1) * dv
    hb = jnp.dot(t, w_ref[...], preferred_element_type=jnp.float32,
                 precision=lax.Precision.HIGHEST)
    hb = (hb + b_ref2[...]) * (g_ref[...] * BN_SCALE) + be_ref[...]
    hb = jnp.maximum(hb, 0.0)
    bv = b_ref[...]
    gids = lax.broadcasted_iota(jnp.int32, (1, ngraphs), 1)
    onehot = (bv == gids).astype(jnp.float32)
    sum_acc[...] += lax.dot_general(
        onehot, hb, (((0,), (0,)), ((), ())),
        preferred_element_type=jnp.float32,
        precision=lax.Precision.HIGHEST)
    cnt_acc[...] += lax.dot_general(
        onehot, jnp.ones((r, 1), jnp.float32), (((0,), (0,)), ((), ())),
        preferred_element_type=jnp.float32)

    gmin = jnp.min(bv)
    gmax = jnp.max(bv)

    def gbody(gid, _):
      m = jnp.where(bv == gid, hb, -jnp.inf)
      mg = jnp.max(m, axis=0, keepdims=True)
      max_acc[pl.ds(gid, 1), :] = jnp.maximum(max_acc[pl.ds(gid, 1), :], mg)
      return 0
    lax.fori_loop(gmin, gmax + 1, gbody, 0)

    @pl.when(i == grid - 1)
    def _():
      mean = sum_acc[...] / jnp.maximum(cnt_acc[...], 1.0)
      mx = max_acc[...]
      mx = jnp.where(jnp.isfinite(mx), mx, 0.0)
      z = jnp.concatenate([mean, mx], axis=1)
      z = gelu(jnp.dot(z, w1[...], preferred_element_type=jnp.float32,
                       precision=lax.Precision.HIGHEST) + v1[...])
      z = gelu(jnp.dot(z, w2[...], preferred_element_type=jnp.float32,
                       precision=lax.Precision.HIGHEST) + v2[...])
      out[...] = (jnp.dot(z, w3[...], preferred_element_type=jnp.float32,
                          precision=lax.Precision.HIGHEST) + v3[...])

  return pl.pallas_call(
      body,
      grid=(grid,),
      in_specs=(
          [pl.BlockSpec((r, din // 2), lambda i: (i, 0))] * 4 +
          [pl.BlockSpec((r, din // 2), lambda i: (i, 0))] * 2 +
          [pl.BlockSpec((r, 1), lambda i: (i, 0)),
           pl.BlockSpec((din, d), lambda i: (0, 0)),
           pl.BlockSpec((1, d), lambda i: (0, 0)),
           pl.BlockSpec((1, d), lambda i: (0, 0)),
           pl.BlockSpec((1, d), lambda i: (0, 0)),
           pl.BlockSpec((r, 1), lambda i: (i, 0)),
           pl.BlockSpec((2 * d, d1), lambda i: (0, 0)),
           pl.BlockSpec((1, d1), lambda i: (0, 0)),
           pl.BlockSpec((d1, d2), lambda i: (0, 0)),
           pl.BlockSpec((1, d2), lambda i: (0, 0)),
           pl.BlockSpec((d2, d3), lambda i: (0, 0)),
           pl.BlockSpec((1, d3), lambda i: (0, 0))]),
      out_specs=pl.BlockSpec((ngraphs, d3), lambda i: (0, 0)),
      out_shape=jax.ShapeDtypeStruct((ngraphs, d3), jnp.float32),
      scratch_shapes=[
          pltpu.VMEM((ngraphs, d), jnp.float32),
          pltpu.VMEM((ngraphs, d), jnp.float32),
          pltpu.VMEM((ngraphs, 1), jnp.float32),
      ],
  )(*a_parts, *p_parts, dinv, w, b.reshape(1, d), g.reshape(1, d),
    be.reshape(1, d), batch.reshape(n, 1), wm1, bm1.reshape(1, d1),
    wm2, bm2.reshape(1, d2), wm3, bm3.reshape(1, d3))


def kernel(x, edge_index, batch, W1, b1, g1, be1, W2, b2, g2, be2,
           W3, b3, g3, be3, Wm1, bm1, Wm2, bm2, Wm3, bm3):
  n, d = x.shape
  e = edge_index.shape[1]
  ngraphs = 64

  src2 = edge_index[0].reshape(e // CHUNK, 1, CHUNK)
  dst2 = edge_index[1].reshape(e // CHUNK, 1, CHUNK)

  deg0, deg1 = _sc_degree(dst2, n)

  dinv, p0 = _tc_prep(x, deg0, deg1)

  a0, a1 = _sc_edge_sum_part(p0, src2, dst2)
  (p1,) = _tc_dense((a0, a1), (p0,), dinv, W1, b1, g1, be1,
                    split_in=False, out_mode='full_p')

  a0, a1 = _sc_edge_sum_part(p1, src2, dst2)
  p2_lo, p2_hi = _tc_dense((a0, a1), (p1,), dinv, W2, b2, g2, be2,
                           split_in=False, out_mode='half_p')

  al0, al1 = _sc_edge_sum_part(p2_lo, src2, dst2)
  ah0, ah1 = _sc_edge_sum_part(p2_hi, src2, dst2)

  return _tc_pool_mlp((al0, al1, ah0, ah1), (p2_lo, p2_hi), dinv,
                      W3, b3, g3, be3, batch, ngraphs,
                      Wm1, bm1, Wm2, bm2, Wm3, bm3)


# speedup vs baseline: 1.0353x; 1.0353x over previous
import functools
import math

import jax
import jax.numpy as jnp
from jax import lax
from jax.experimental import pallas as pl
from jax.experimental.pallas import tpu as pltpu
from jax.experimental.pallas import tpu_sc as plsc

NC = 2
NS = 16
CHUNK = 128

BN_SCALE = 1.0 / math.sqrt(1.0 + 1e-5)


def _sc_mesh():
  return plsc.VectorSubcoreMesh(
      core_axis_name="c", subcore_axis_name="s", num_cores=NC,
      num_subcores=NS)


def _sc_degree(dst2, n):
  nchunks = dst2.shape[0]
  pc = nchunks // NC
  pt = pc // NS
  extra = pc - pt * NS
  rows_t = (n // NS) // 8 * 8
  rows_tail = n - NS * rows_t

  GS_D = 26
  WAVE = 13
  assert pt % GS_D == 0 and GS_D % WAVE == 0

  def body(dst_hbm, out0, out1, didx, xidx_v, ones_v, stage_v, acc_sh, sem):
    c = lax.axis_index("c")
    s = lax.axis_index("s")
    base = c * pc + s * pt
    xidx_ref = xidx_v.at[0, 0]

    def setones(i, _):
      ones_v[pl.ds(i * 16, 16)] = jnp.ones((16,), jnp.float32)
      stage_v[pl.ds(i * 16, 16)] = jnp.zeros((16,), jnp.float32)
      return 0
    lax.fori_loop(0, CHUNK // 16, setones, 0, unroll=True)

    def zinit(i, _):
      pltpu.sync_copy(stage_v,
                      acc_sh.at[pl.ds(s * rows_t + i * CHUNK, CHUNK)])
      return 0
    lax.fori_loop(0, rows_t // CHUNK, zinit, 0)
    rem = rows_t - (rows_t // CHUNK) * CHUNK
    if rem:
      pltpu.sync_copy(stage_v.at[pl.ds(0, rem)],
                      acc_sh.at[pl.ds(s * rows_t + rows_t - rem, rem)])

    @pl.when(s == 0)
    def _():
      pltpu.sync_copy(stage_v.at[pl.ds(0, rows_tail)],
                      acc_sh.at[pl.ds(NS * rows_t, rows_tail)])

    plsc.subcore_barrier()

    def group(gd, _):
      pltpu.sync_copy(dst_hbm.at[pl.ds(base + gd * GS_D, GS_D)], didx)

      def wave(wv, _):
        for b in range(WAVE):
          pltpu.async_copy(ones_v, acc_sh.at[didx.at[wv * WAVE + b, 0]],
                           sem, add=True)
        for b in range(WAVE):
          pltpu.make_async_copy(ones_v, acc_sh.at[didx.at[0, 0]], sem).wait()
        return 0
      lax.fori_loop(0, GS_D // WAVE, wave, 0)
      return 0
    lax.fori_loop(0, pt // GS_D, group, 0)

    @pl.when(s < extra)
    def _():
      pltpu.sync_copy(dst_hbm.at[pl.ds(c * pc + pt * NS + s, 1)], xidx_v)
      pltpu.sync_copy(ones_v, acc_sh.at[xidx_ref], add=True)

    plsc.subcore_barrier()

    def writeout(out_hbm):
      def wchunk(i, _):
        pltpu.sync_copy(acc_sh.at[pl.ds(s * rows_t + i * CHUNK, CHUNK)],
                        stage_v)
        pltpu.sync_copy(stage_v,
                        out_hbm.at[pl.ds(s * rows_t + i * CHUNK, CHUNK)])
        return 0
      lax.fori_loop(0, rows_t // CHUNK, wchunk, 0)
      if rem:
        pltpu.sync_copy(acc_sh.at[pl.ds(s * rows_t + rows_t - rem, rem)],
                        stage_v.at[pl.ds(0, rem)])
        pltpu.sync_copy(stage_v.at[pl.ds(0, rem)],
                        out_hbm.at[pl.ds(s * rows_t + rows_t - rem, rem)])

      @pl.when(s == 0)
      def _():
        pltpu.sync_copy(acc_sh.at[pl.ds(NS * rows_t, rows_tail)],
                        stage_v.at[pl.ds(0, rows_tail)])
        pltpu.sync_copy(stage_v.at[pl.ds(0, rows_tail)],
                        out_hbm.at[pl.ds(NS * rows_t, rows_tail)])

    @pl.when(c == 0)
    def _():
      writeout(out0)

    @pl.when(c == 1)
    def _():
      writeout(out1)

  f = pl.kernel(
      body,
      out_type=[jax.ShapeDtypeStruct((n,), jnp.float32),
                jax.ShapeDtypeStruct((n,), jnp.float32)],
      mesh=_sc_mesh(),
      scratch_types=[
          pltpu.VMEM((GS_D, 1, CHUNK), jnp.int32),
          pltpu.VMEM((1, 1, CHUNK), jnp.int32),
          pltpu.VMEM((CHUNK,), jnp.float32),
          pltpu.VMEM((CHUNK,), jnp.float32),
          pltpu.VMEM_SHARED((n,), jnp.float32),
          pltpu.SemaphoreType.DMA,
      ],
  )
  return f(dst2)


def _sc_edge_sum_part(p, src2, dst2):
  n, d = p.shape
  nchunks = src2.shape[0]
  pc = nchunks // NC
  pt = pc // NS
  extra = pc - pt * NS
  GS = 26
  assert pt % GS == 0
  rows_t = (n // NS) // 8 * 8
  rows_tail = n - NS * rows_t
  sub = [CHUNK] * (rows_t // CHUNK)
  if rows_t % CHUNK:
    sub.append(rows_t % CHUNK)

  def body(p_hbm, src_hbm, dst_hbm, out0, out1,
           sidx, didx, sidx2, didx2, xsidx, xdidx, rows_v, rows_w,
           acc_sh, sem_a, sem_b, sem_sa, sem_sb, sem_ia, sem_ib):
    c = lax.axis_index("c")
    s = lax.axis_index("s")
    base = c * pc + s * pt
    xsidx_ref = xsidx.at[0, 0]
    xdidx_ref = xdidx.at[0, 0]

    def zrow(i, _):
      row = rows_v.at[i]
      def zcol(j, _):
        row[pl.ds(j * 16, 16)] = jnp.zeros((16,), jnp.float32)
        return 0
      lax.fori_loop(0, d // 16, zcol, 0, unroll=True)
      return 0
    lax.fori_loop(0, CHUNK, zrow, 0)

    off = 0
    for sz in sub:
      pltpu.sync_copy(rows_v.at[pl.ds(0, sz)],
                      acc_sh.at[pl.ds(s * rows_t + off, sz)])
      off += sz

    @pl.when(s == 0)
    def _():
      pltpu.sync_copy(rows_v.at[pl.ds(0, rows_tail)],
                      acc_sh.at[pl.ds(NS * rows_t, rows_tail)])

    plsc.subcore_barrier()

    ngroups = pt // GS
    sbufs = [sidx, sidx2]
    dbufs = [didx, didx2]
    pltpu.async_copy(src_hbm.at[pl.ds(base, GS)], sidx, sem_ia)
    pltpu.async_copy(dst_hbm.at[pl.ds(base, GS)], didx, sem_ib)

    for gg in range(ngroups):
      si = sbufs[gg % 2]
      di = dbufs[gg % 2]
      pltpu.make_async_copy(src_hbm.at[pl.ds(base, GS)], si, sem_ia).wait()
      pltpu.make_async_copy(dst_hbm.at[pl.ds(base, GS)], di, sem_ib).wait()
      if gg + 1 < ngroups:
        nb = base + (gg + 1) * GS
        pltpu.async_copy(src_hbm.at[pl.ds(nb, GS)], sbufs[(gg + 1) % 2],
                         sem_ia)
        pltpu.async_copy(dst_hbm.at[pl.ds(nb, GS)], dbufs[(gg + 1) % 2],
                         sem_ib)
      pltpu.async_copy(p_hbm.at[si.at[0, 0]], rows_v, sem_a)
      pltpu.async_copy(p_hbm.at[si.at[1, 0]], rows_w, sem_b)

      def pair(k, _):
        j0 = 2 * k
        pltpu.make_async_copy(p_hbm.at[si.at[0, 0]], rows_v, sem_a).wait()
        ca = pltpu.async_copy(rows_v, acc_sh.at[di.at[j0, 0]], sem_sa,
                              add=True)
        pltpu.make_async_copy(p_hbm.at[si.at[0, 0]], rows_w, sem_b).wait()
        cb = pltpu.async_copy(rows_w, acc_sh.at[di.at[j0 + 1, 0]], sem_sb,
                              add=True)
        ca.wait()

        @pl.when(j0 + 2 < GS)
        def _():
          pltpu.async_copy(p_hbm.at[si.at[j0 + 2, 0]], rows_v, sem_a)

        cb.wait()

        @pl.when(j0 + 3 < GS)
        def _():
          pltpu.async_copy(p_hbm.at[si.at[j0 + 3, 0]], rows_w, sem_b)

        return 0
      lax.fori_loop(0, GS // 2, pair, 0)

    @pl.when(s < extra)
    def _():
      pltpu.sync_copy(src_hbm.at[pl.ds(c * pc + pt * NS + s, 1)], xsidx)
      pltpu.sync_copy(dst_hbm.at[pl.ds(c * pc + pt * NS + s, 1)], xdidx)
      pltpu.async_copy(p_hbm.at[xsidx_ref], rows_v, sem_a).wait()
      pltpu.sync_copy(rows_v, acc_sh.at[xdidx_ref], add=True)

    plsc.subcore_barrier()

    def writeout(out_hbm):
      woff = 0
      for sz in sub:
        pltpu.sync_copy(acc_sh.at[pl.ds(s * rows_t + woff, sz)],
                        rows_v.at[pl.ds(0, sz)])
        pltpu.sync_copy(rows_v.at[pl.ds(0, sz)],
                        out_hbm.at[pl.ds(s * rows_t + woff, sz)])
        woff += sz

      @pl.when(s == 0)
      def _():
        pltpu.sync_copy(acc_sh.at[pl.ds(NS * rows_t, rows_tail)],
                        rows_v.at[pl.ds(0, rows_tail)])
        pltpu.sync_copy(rows_v.at[pl.ds(0, rows_tail)],
                        out_hbm.at[pl.ds(NS * rows_t, rows_tail)])

    @pl.when(c == 0)
    def _():
      writeout(out0)

    @pl.when(c == 1)
    def _():
      writeout(out1)

  f = pl.kernel(
      body,
      out_type=[jax.ShapeDtypeStruct((n, d), jnp.float32),
                jax.ShapeDtypeStruct((n, d), jnp.float32)],
      mesh=_sc_mesh(),
      scratch_types=[
          pltpu.VMEM((GS, 1, CHUNK), jnp.int32),
          pltpu.VMEM((GS, 1, CHUNK), jnp.int32),
          pltpu.VMEM((GS, 1, CHUNK), jnp.int32),
          pltpu.VMEM((GS, 1, CHUNK), jnp.int32),
          pltpu.VMEM((1, 1, CHUNK), jnp.int32),
          pltpu.VMEM((1, 1, CHUNK), jnp.int32),
          pltpu.VMEM((CHUNK, d), jnp.float32),
          pltpu.VMEM((CHUNK, d), jnp.float32),
          pltpu.VMEM_SHARED((n, d), jnp.float32),
          pltpu.SemaphoreType.DMA,
          pltpu.SemaphoreType.DMA,
          pltpu.SemaphoreType.DMA,
          pltpu.SemaphoreType.DMA,
          pltpu.SemaphoreType.DMA,
          pltpu.SemaphoreType.DMA,
      ],
  )
  return f(p, src2, dst2)


def _tc_prep(x, deg0, deg1):
  n, d = x.shape
  r = 1000
  grid = n // r

  def body(x_ref, d0_ref, d1_ref, di_ref, p_ref):
    deg = d0_ref[...] + d1_ref[...] + 1.0
    di = lax.rsqrt(deg)
    di_ref[...] = di
    p_ref[...] = x_ref[...] * di

  return pl.pallas_call(
      body,
      grid=(grid,),
      in_specs=[
          pl.BlockSpec((r, d), lambda i: (i, 0)),
          pl.BlockSpec((r, 1), lambda i: (i, 0)),
          pl.BlockSpec((r, 1), lambda i: (i, 0)),
      ],
      out_specs=[
          pl.BlockSpec((r, 1), lambda i: (i, 0)),
          pl.BlockSpec((r, d), lambda i: (i, 0)),
      ],
      out_shape=[
          jax.ShapeDtypeStruct((n, 1), jnp.float32),
          jax.ShapeDtypeStruct((n, d), jnp.float32),
      ],
  )(x, deg0.reshape(n, 1), deg1.reshape(n, 1))


def _tc_dense(a_parts, p_parts, dinv, w, b, g, be, split_in, out_mode):
  n = a_parts[0].shape[0]
  din = w.shape[0]
  dout = w.shape[1]
  r = 1000
  grid = n // r

  def body(*refs):
    a_refs = refs[:len(a_parts)]
    p_refs = refs[len(a_parts):len(a_parts) + len(p_parts)]
    di, w_ref, b_ref, g_ref, be_ref = refs[len(a_parts) + len(p_parts):-
                                           (2 if out_mode == 'half_p' else 1)]
    outs = refs[-(2 if out_mode == 'half_p' else 1):]
    dv = di[...]
    def up(r_):
      return r_[...].astype(jnp.float32)
    if split_in:
      t = jnp.concatenate(
          [up(a_refs[0]) + up(a_refs[1]) + up(p_refs[0]),
           up(a_refs[2]) + up(a_refs[3]) + up(p_refs[1])], axis=1) * dv
    else:
      t = (up(a_refs[0]) + up(a_refs[1]) + up(p_refs[0])) * dv
    h = jnp.dot(t, w_ref[...], preferred_element_type=jnp.float32)
    h = (h + b_ref[...]) * (g_ref[...] * BN_SCALE) + be_ref[...]
    h = jnp.maximum(h, 0.0)
    if out_mode == 'h':
      outs[0][...] = h
    elif out_mode == 'full_p':
      outs[0][...] = h * dv
    else:
      pn = h * dv
      outs[0][...] = pn[:, :dout // 2]
      outs[1][...] = pn[:, dout // 2:]

  out_dt = jnp.float32
  if out_mode == 'half_p':
    out_specs = [pl.BlockSpec((r, dout // 2), lambda i: (i, 0)),
                 pl.BlockSpec((r, dout // 2), lambda i: (i, 0))]
    out_shape = [jax.ShapeDtypeStruct((n, dout // 2), out_dt),
                 jax.ShapeDtypeStruct((n, dout // 2), out_dt)]
  else:
    out_specs = [pl.BlockSpec((r, dout), lambda i: (i, 0))]
    out_shape = [jax.ShapeDtypeStruct((n, dout), out_dt)]

  in_specs = (
      [pl.BlockSpec((r, a.shape[1]), lambda i: (i, 0)) for a in a_parts] +
      [pl.BlockSpec((r, p.shape[1]), lambda i: (i, 0)) for p in p_parts] +
      [pl.BlockSpec((r, 1), lambda i: (i, 0)),
       pl.BlockSpec((din, dout), lambda i: (0, 0)),
       pl.BlockSpec((1, dout), lambda i: (0, 0)),
       pl.BlockSpec((1, dout), lambda i: (0, 0)),
       pl.BlockSpec((1, dout), lambda i: (0, 0))])

  res = pl.pallas_call(
      body,
      grid=(grid,),
      in_specs=in_specs,
      out_specs=out_specs,
      out_shape=out_shape,
  )(*a_parts, *p_parts, dinv,
    w, b.reshape(1, dout), g.reshape(1, dout), be.reshape(1, dout))
  return res


def _tc_pool_mlp(a_parts, p_parts, dinv, w, b, g, be, batch, ngraphs,
                 wm1, bm1, wm2, bm2, wm3, bm3):
  n = a_parts[0].shape[0]
  din = w.shape[0]
  d = w.shape[1]
  r = 1000
  grid = n // r
  d1 = wm1.shape[1]
  d2 = wm2.shape[1]
  d3 = wm3.shape[1]

  def gelu(z):
    return 0.5 * z * (1.0 + lax.erf(z * (1.0 / math.sqrt(2.0))))

  def body(al0, al1, ah0, ah1, plo, phi, di, w_ref, b_ref2, g_ref, be_ref,
           b_ref, w1, v1, w2, v2, w3, v3, out,
           sum_acc, max_acc, cnt_acc):
    i = pl.program_id(0)

    @pl.when(i == 0)
    def _():
      sum_acc[...] = jnp.zeros_like(sum_acc)
      cnt_acc[...] = jnp.zeros_like(cnt_acc)
      max_acc[...] = jnp.full_like(max_acc, -jnp.inf)

    dv = di[...]
    def up(r_):
      return r_[...].astype(jnp.float32)
    t = jnp.concatenate([up(al0) + up(al1) + up(plo),
                         up(ah0) + up(ah1) + up(phi)], axis=1) * dv
    hb = jnp.dot(t, w_ref[...], preferred_element_type=jnp.float32)
    hb = (hb + b_ref2[...]) * (g_ref[...] * BN_SCALE) + be_ref[...]
    hb = jnp.maximum(hb, 0.0)
    bv = b_ref[...]
    gids = lax.broadcasted_iota(jnp.int32, (1, ngraphs), 1)
    onehot = (bv == gids).astype(jnp.float32)
    sum_acc[...] += lax.dot_general(
        onehot, hb, (((0,), (0,)), ((), ())),
        preferred_element_type=jnp.float32)
    cnt_acc[...] += lax.dot_general(
        onehot, jnp.ones((r, 1), jnp.float32), (((0,), (0,)), ((), ())),
        preferred_element_type=jnp.float32)

    gmin = jnp.min(bv)
    gmax = jnp.max(bv)

    def gbody(gid, _):
      m = jnp.where(bv == gid, hb, -jnp.inf)
      mg = jnp.max(m, axis=0, keepdims=True)
      max_acc[pl.ds(gid, 1), :] = jnp.maximum(max_acc[pl.ds(gid, 1), :], mg)
      return 0
    lax.fori_loop(gmin, gmax + 1, gbody, 0)

    @pl.when(i == grid - 1)
    def _():
      mean = sum_acc[...] / jnp.maximum(cnt_acc[...], 1.0)
      mx = max_acc[...]
      mx = jnp.where(jnp.isfinite(mx), mx, 0.0)
      z = jnp.concatenate([mean, mx], axis=1)
      z = gelu(jnp.dot(z, w1[...], preferred_element_type=jnp.float32)
               + v1[...])
      z = gelu(jnp.dot(z, w2[...], preferred_element_type=jnp.float32)
               + v2[...])
      out[...] = (jnp.dot(z, w3[...], preferred_element_type=jnp.float32)
                  + v3[...])

  return pl.pallas_call(
      body,
      grid=(grid,),
      in_specs=(
          [pl.BlockSpec((r, din // 2), lambda i: (i, 0))] * 4 +
          [pl.BlockSpec((r, din // 2), lambda i: (i, 0))] * 2 +
          [pl.BlockSpec((r, 1), lambda i: (i, 0)),
           pl.BlockSpec((din, d), lambda i: (0, 0)),
           pl.BlockSpec((1, d), lambda i: (0, 0)),
           pl.BlockSpec((1, d), lambda i: (0, 0)),
           pl.BlockSpec((1, d), lambda i: (0, 0)),
           pl.BlockSpec((r, 1), lambda i: (i, 0)),
           pl.BlockSpec((2 * d, d1), lambda i: (0, 0)),
           pl.BlockSpec((1, d1), lambda i: (0, 0)),
           pl.BlockSpec((d1, d2), lambda i: (0, 0)),
           pl.BlockSpec((1, d2), lambda i: (0, 0)),
           pl.BlockSpec((d2, d3), lambda i: (0, 0)),
           pl.BlockSpec((1, d3), lambda i: (0, 0))]),
      out_specs=pl.BlockSpec((ngraphs, d3), lambda i: (0, 0)),
      out_shape=jax.ShapeDtypeStruct((ngraphs, d3), jnp.float32),
      scratch_shapes=[
          pltpu.VMEM((ngraphs, d), jnp.float32),
          pltpu.VMEM((ngraphs, d), jnp.float32),
          pltpu.VMEM((ngraphs, 1), jnp.float32),
      ],
  )(*a_parts, *p_parts, dinv, w, b.reshape(1, d), g.reshape(1, d),
    be.reshape(1, d), batch.reshape(n, 1), wm1, bm1.reshape(1, d1),
    wm2, bm2.reshape(1, d2), wm3, bm3.reshape(1, d3))


def kernel(x, edge_index, batch, W1, b1, g1, be1, W2, b2, g2, be2,
           W3, b3, g3, be3, Wm1, bm1, Wm2, bm2, Wm3, bm3):
  n, d = x.shape
  e = edge_index.shape[1]
  ngraphs = 64

  src2 = edge_index[0].reshape(e // CHUNK, 1, CHUNK)
  dst2 = edge_index[1].reshape(e // CHUNK, 1, CHUNK)

  deg0, deg1 = _sc_degree(dst2, n)

  dinv, p0 = _tc_prep(x, deg0, deg1)

  a0, a1 = _sc_edge_sum_part(p0, src2, dst2)
  (p1,) = _tc_dense((a0, a1), (p0,), dinv, W1, b1, g1, be1,
                    split_in=False, out_mode='full_p')

  a0, a1 = _sc_edge_sum_part(p1, src2, dst2)
  p2_lo, p2_hi = _tc_dense((a0, a1), (p1,), dinv, W2, b2, g2, be2,
                           split_in=False, out_mode='half_p')

  al0, al1 = _sc_edge_sum_part(p2_lo, src2, dst2)
  ah0, ah1 = _sc_edge_sum_part(p2_hi, src2, dst2)

  return _tc_pool_mlp((al0, al1, ah0, ah1), (p2_lo, p2_hi), dinv,
                      W3, b3, g3, be3, batch, ngraphs,
                      Wm1, bm1, Wm2, bm2, Wm3, bm3)


# speedup vs baseline: 1.0392x; 1.0038x over previous
import functools
import math

import jax
import jax.numpy as jnp
from jax import lax
from jax.experimental import pallas as pl
from jax.experimental.pallas import tpu as pltpu
from jax.experimental.pallas import tpu_sc as plsc

NC = 2
NS = 16
CHUNK = 128

BN_SCALE = 1.0 / math.sqrt(1.0 + 1e-5)


def _sc_mesh():
  return plsc.VectorSubcoreMesh(
      core_axis_name="c", subcore_axis_name="s", num_cores=NC,
      num_subcores=NS)


def _sc_degree(dst2, n):
  nchunks = dst2.shape[0]
  pc = nchunks // NC
  pt = pc // NS
  extra = pc - pt * NS
  rows_t = (n // NS) // 8 * 8
  rows_tail = n - NS * rows_t

  GS_D = 26
  WAVE = 13
  assert pt % GS_D == 0 and GS_D % WAVE == 0

  def body(dst_hbm, out0, out1, didx, xidx_v, ones_v, stage_v, acc_sh, sem):
    c = lax.axis_index("c")
    s = lax.axis_index("s")
    base = c * pc + s * pt
    xidx_ref = xidx_v.at[0, 0]

    def setones(i, _):
      ones_v[pl.ds(i * 16, 16)] = jnp.ones((16,), jnp.float32)
      stage_v[pl.ds(i * 16, 16)] = jnp.zeros((16,), jnp.float32)
      return 0
    lax.fori_loop(0, CHUNK // 16, setones, 0, unroll=True)

    def zinit(i, _):
      pltpu.sync_copy(stage_v,
                      acc_sh.at[pl.ds(s * rows_t + i * CHUNK, CHUNK)])
      return 0
    lax.fori_loop(0, rows_t // CHUNK, zinit, 0)
    rem = rows_t - (rows_t // CHUNK) * CHUNK
    if rem:
      pltpu.sync_copy(stage_v.at[pl.ds(0, rem)],
                      acc_sh.at[pl.ds(s * rows_t + rows_t - rem, rem)])

    @pl.when(s == 0)
    def _():
      pltpu.sync_copy(stage_v.at[pl.ds(0, rows_tail)],
                      acc_sh.at[pl.ds(NS * rows_t, rows_tail)])

    plsc.subcore_barrier()

    def group(gd, _):
      pltpu.sync_copy(dst_hbm.at[pl.ds(base + gd * GS_D, GS_D)], didx)

      def wave(wv, _):
        for b in range(WAVE):
          pltpu.async_copy(ones_v, acc_sh.at[didx.at[wv * WAVE + b, 0]],
                           sem, add=True)
        for b in range(WAVE):
          pltpu.make_async_copy(ones_v, acc_sh.at[didx.at[0, 0]], sem).wait()
        return 0
      lax.fori_loop(0, GS_D // WAVE, wave, 0)
      return 0
    lax.fori_loop(0, pt // GS_D, group, 0)

    @pl.when(s < extra)
    def _():
      pltpu.sync_copy(dst_hbm.at[pl.ds(c * pc + pt * NS + s, 1)], xidx_v)
      pltpu.sync_copy(ones_v, acc_sh.at[xidx_ref], add=True)

    plsc.subcore_barrier()

    def writeout(out_hbm):
      def wchunk(i, _):
        pltpu.sync_copy(acc_sh.at[pl.ds(s * rows_t + i * CHUNK, CHUNK)],
                        stage_v)
        pltpu.sync_copy(stage_v,
                        out_hbm.at[pl.ds(s * rows_t + i * CHUNK, CHUNK)])
        return 0
      lax.fori_loop(0, rows_t // CHUNK, wchunk, 0)
      if rem:
        pltpu.sync_copy(acc_sh.at[pl.ds(s * rows_t + rows_t - rem, rem)],
                        stage_v.at[pl.ds(0, rem)])
        pltpu.sync_copy(stage_v.at[pl.ds(0, rem)],
                        out_hbm.at[pl.ds(s * rows_t + rows_t - rem, rem)])

      @pl.when(s == 0)
      def _():
        pltpu.sync_copy(acc_sh.at[pl.ds(NS * rows_t, rows_tail)],
                        stage_v.at[pl.ds(0, rows_tail)])
        pltpu.sync_copy(stage_v.at[pl.ds(0, rows_tail)],
                        out_hbm.at[pl.ds(NS * rows_t, rows_tail)])

    @pl.when(c == 0)
    def _():
      writeout(out0)

    @pl.when(c == 1)
    def _():
      writeout(out1)

  f = pl.kernel(
      body,
      out_type=[jax.ShapeDtypeStruct((n,), jnp.float32),
                jax.ShapeDtypeStruct((n,), jnp.float32)],
      mesh=_sc_mesh(),
      scratch_types=[
          pltpu.VMEM((GS_D, 1, CHUNK), jnp.int32),
          pltpu.VMEM((1, 1, CHUNK), jnp.int32),
          pltpu.VMEM((CHUNK,), jnp.float32),
          pltpu.VMEM((CHUNK,), jnp.float32),
          pltpu.VMEM_SHARED((n,), jnp.float32),
          pltpu.SemaphoreType.DMA,
      ],
  )
  return f(dst2)


def _sc_edge_sum_part(p, src2, dst2):
  n, d = p.shape
  nchunks = src2.shape[0]
  pc = nchunks // NC
  pt = pc // NS
  extra = pc - pt * NS
  GS = 26
  assert pt % GS == 0
  rows_t = (n // NS) // 8 * 8
  rows_tail = n - NS * rows_t
  sub = [CHUNK] * (rows_t // CHUNK)
  if rows_t % CHUNK:
    sub.append(rows_t % CHUNK)

  def body(p_hbm, src_hbm, dst_hbm, out0, out1,
           sidx, didx, sidx2, didx2, xsidx, xdidx, rows_v, rows_w,
           acc_sh, sem_a, sem_b, sem_sa, sem_sb, sem_ia, sem_ib):
    c = lax.axis_index("c")
    s = lax.axis_index("s")
    base = c * pc + s * pt
    xsidx_ref = xsidx.at[0, 0]
    xdidx_ref = xdidx.at[0, 0]

    def zrow(i, _):
      row = rows_v.at[i]
      def zcol(j, _):
        row[pl.ds(j * 16, 16)] = jnp.zeros((16,), jnp.float32)
        return 0
      lax.fori_loop(0, d // 16, zcol, 0, unroll=True)
      return 0
    lax.fori_loop(0, CHUNK, zrow, 0)

    off = 0
    for sz in sub:
      pltpu.sync_copy(rows_v.at[pl.ds(0, sz)],
                      acc_sh.at[pl.ds(s * rows_t + off, sz)])
      off += sz

    @pl.when(s == 0)
    def _():
      pltpu.sync_copy(rows_v.at[pl.ds(0, rows_tail)],
                      acc_sh.at[pl.ds(NS * rows_t, rows_tail)])

    plsc.subcore_barrier()

    ngroups = pt // GS
    sbufs = [sidx, sidx2]
    dbufs = [didx, didx2]
    pltpu.async_copy(src_hbm.at[pl.ds(base, GS)], sidx, sem_ia)
    pltpu.async_copy(dst_hbm.at[pl.ds(base, GS)], didx, sem_ib)

    for gg in range(ngroups):
      si = sbufs[gg % 2]
      di = dbufs[gg % 2]
      pltpu.make_async_copy(src_hbm.at[pl.ds(base, GS)], si, sem_ia).wait()
      pltpu.make_async_copy(dst_hbm.at[pl.ds(base, GS)], di, sem_ib).wait()
      if gg + 1 < ngroups:
        nb = base + (gg + 1) * GS
        pltpu.async_copy(src_hbm.at[pl.ds(nb, GS)], sbufs[(gg + 1) % 2],
                         sem_ia)
        pltpu.async_copy(dst_hbm.at[pl.ds(nb, GS)], dbufs[(gg + 1) % 2],
                         sem_ib)
      pltpu.async_copy(p_hbm.at[si.at[0, 0]], rows_v, sem_a)
      pltpu.async_copy(p_hbm.at[si.at[1, 0]], rows_w, sem_b)

      def pair(k, _):
        j0 = 2 * k
        pltpu.make_async_copy(p_hbm.at[si.at[0, 0]], rows_v, sem_a).wait()
        ca = pltpu.async_copy(rows_v, acc_sh.at[di.at[j0, 0]], sem_sa,
                              add=True)
        pltpu.make_async_copy(p_hbm.at[si.at[0, 0]], rows_w, sem_b).wait()
        cb = pltpu.async_copy(rows_w, acc_sh.at[di.at[j0 + 1, 0]], sem_sb,
                              add=True)
        ca.wait()

        @pl.when(j0 + 2 < GS)
        def _():
          pltpu.async_copy(p_hbm.at[si.at[j0 + 2, 0]], rows_v, sem_a)

        cb.wait()

        @pl.when(j0 + 3 < GS)
        def _():
          pltpu.async_copy(p_hbm.at[si.at[j0 + 3, 0]], rows_w, sem_b)

        return 0
      lax.fori_loop(0, GS // 2, pair, 0)

    @pl.when(s < extra)
    def _():
      pltpu.sync_copy(src_hbm.at[pl.ds(c * pc + pt * NS + s, 1)], xsidx)
      pltpu.sync_copy(dst_hbm.at[pl.ds(c * pc + pt * NS + s, 1)], xdidx)
      pltpu.async_copy(p_hbm.at[xsidx_ref], rows_v, sem_a).wait()
      pltpu.sync_copy(rows_v, acc_sh.at[xdidx_ref], add=True)

    plsc.subcore_barrier()

    def writeout(out_hbm):
      woff = 0
      for sz in sub:
        pltpu.sync_copy(acc_sh.at[pl.ds(s * rows_t + woff, sz)],
                        rows_v.at[pl.ds(0, sz)])
        pltpu.sync_copy(rows_v.at[pl.ds(0, sz)],
                        out_hbm.at[pl.ds(s * rows_t + woff, sz)])
        woff += sz

      @pl.when(s == 0)
      def _():
        pltpu.sync_copy(acc_sh.at[pl.ds(NS * rows_t, rows_tail)],
                        rows_v.at[pl.ds(0, rows_tail)])
        pltpu.sync_copy(rows_v.at[pl.ds(0, rows_tail)],
                        out_hbm.at[pl.ds(NS * rows_t, rows_tail)])

    @pl.when(c == 0)
    def _():
      writeout(out0)

    @pl.when(c == 1)
    def _():
      writeout(out1)

  f = pl.kernel(
      body,
      out_type=[jax.ShapeDtypeStruct((n, d), jnp.float32),
                jax.ShapeDtypeStruct((n, d), jnp.float32)],
      mesh=_sc_mesh(),
      scratch_types=[
          pltpu.VMEM((GS, 1, CHUNK), jnp.int32),
          pltpu.VMEM((GS, 1, CHUNK), jnp.int32),
          pltpu.VMEM((GS, 1, CHUNK), jnp.int32),
          pltpu.VMEM((GS, 1, CHUNK), jnp.int32),
          pltpu.VMEM((1, 1, CHUNK), jnp.int32),
          pltpu.VMEM((1, 1, CHUNK), jnp.int32),
          pltpu.VMEM((CHUNK, d), jnp.float32),
          pltpu.VMEM((CHUNK, d), jnp.float32),
          pltpu.VMEM_SHARED((n, d), jnp.float32),
          pltpu.SemaphoreType.DMA,
          pltpu.SemaphoreType.DMA,
          pltpu.SemaphoreType.DMA,
          pltpu.SemaphoreType.DMA,
          pltpu.SemaphoreType.DMA,
          pltpu.SemaphoreType.DMA,
      ],
  )
  return f(p, src2, dst2)


def _tc_prep(x, deg0, deg1, w1):
  n, d = x.shape
  dout = w1.shape[1]
  r = 1000
  grid = n // r

  def body(x_ref, d0_ref, d1_ref, w_ref, di_ref, q_ref):
    deg = d0_ref[...] + d1_ref[...] + 1.0
    di = lax.rsqrt(deg)
    di_ref[...] = di
    m = jnp.dot(x_ref[...], w_ref[...], preferred_element_type=jnp.float32)
    q_ref[...] = m * di

  return pl.pallas_call(
      body,
      grid=(grid,),
      in_specs=[
          pl.BlockSpec((r, d), lambda i: (i, 0)),
          pl.BlockSpec((r, 1), lambda i: (i, 0)),
          pl.BlockSpec((r, 1), lambda i: (i, 0)),
          pl.BlockSpec((d, dout), lambda i: (0, 0)),
      ],
      out_specs=[
          pl.BlockSpec((r, 1), lambda i: (i, 0)),
          pl.BlockSpec((r, dout), lambda i: (i, 0)),
      ],
      out_shape=[
          jax.ShapeDtypeStruct((n, 1), jnp.float32),
          jax.ShapeDtypeStruct((n, dout), jnp.float32),
      ],
  )(x, deg0.reshape(n, 1), deg1.reshape(n, 1), w1)


def _tc_post1(a0, a1, q1, dinv, b, g, be):
  n, d = q1.shape
  r = 1000
  grid = n // r

  def body(a0r, a1r, qr, di, b_ref, g_ref, be_ref, out):
    dv = di[...]
    t = (a0r[...] + a1r[...] + qr[...]) * dv
    h = (t + b_ref[...]) * (g_ref[...] * BN_SCALE) + be_ref[...]
    h = jnp.maximum(h, 0.0)
    out[...] = h * dv

  return pl.pallas_call(
      body,
      grid=(grid,),
      in_specs=[
          pl.BlockSpec((r, d), lambda i: (i, 0)),
          pl.BlockSpec((r, d), lambda i: (i, 0)),
          pl.BlockSpec((r, d), lambda i: (i, 0)),
          pl.BlockSpec((r, 1), lambda i: (i, 0)),
          pl.BlockSpec((1, d), lambda i: (0, 0)),
          pl.BlockSpec((1, d), lambda i: (0, 0)),
          pl.BlockSpec((1, d), lambda i: (0, 0)),
      ],
      out_specs=pl.BlockSpec((r, d), lambda i: (i, 0)),
      out_shape=jax.ShapeDtypeStruct((n, d), jnp.float32),
  )(a0, a1, q1, dinv, b.reshape(1, d), g.reshape(1, d), be.reshape(1, d))


def _tc_dense2(a0, a1, p2, dinv, w2, b, g, be, w3):
  n, d = p2.shape
  d2 = w2.shape[1]
  d3 = w3.shape[1]
  r = 1000
  grid = n // r

  def body(a0r, a1r, pr, di, w2r, b_ref, g_ref, be_ref, w3r, olo, ohi):
    dv = di[...]
    t = (a0r[...] + a1r[...] + pr[...]) * dv
    h = jnp.dot(t, w2r[...], preferred_element_type=jnp.float32)
    h = (h + b_ref[...]) * (g_ref[...] * BN_SCALE) + be_ref[...]
    h = jnp.maximum(h, 0.0)
    q = jnp.dot(h, w3r[...], preferred_element_type=jnp.float32) * dv
    olo[...] = q[:, :d3 // 2]
    ohi[...] = q[:, d3 // 2:]

  return pl.pallas_call(
      body,
      grid=(grid,),
      in_specs=[
          pl.BlockSpec((r, d), lambda i: (i, 0)),
          pl.BlockSpec((r, d), lambda i: (i, 0)),
          pl.BlockSpec((r, d), lambda i: (i, 0)),
          pl.BlockSpec((r, 1), lambda i: (i, 0)),
          pl.BlockSpec((d, d2), lambda i: (0, 0)),
          pl.BlockSpec((1, d2), lambda i: (0, 0)),
          pl.BlockSpec((1, d2), lambda i: (0, 0)),
          pl.BlockSpec((1, d2), lambda i: (0, 0)),
          pl.BlockSpec((d2, d3), lambda i: (0, 0)),
      ],
      out_specs=[pl.BlockSpec((r, d3 // 2), lambda i: (i, 0)),
                 pl.BlockSpec((r, d3 // 2), lambda i: (i, 0))],
      out_shape=[jax.ShapeDtypeStruct((n, d3 // 2), jnp.float32),
                 jax.ShapeDtypeStruct((n, d3 // 2), jnp.float32)],
  )(a0, a1, p2, dinv, w2, b.reshape(1, d2), g.reshape(1, d2),
    be.reshape(1, d2), w3)


def _tc_pool_mlp(a_parts, q_parts, dinv, b, g, be, batch, ngraphs,
                 wm1, bm1, wm2, bm2, wm3, bm3):
  n = a_parts[0].shape[0]
  d = 2 * q_parts[0].shape[1]
  r = 1000
  grid = n // r
  d1 = wm1.shape[1]
  d2 = wm2.shape[1]
  d3 = wm3.shape[1]

  def gelu(z):
    return 0.5 * z * (1.0 + lax.erf(z * (1.0 / math.sqrt(2.0))))

  def body(al0, al1, ah0, ah1, qlo, qhi, di, b_ref2, g_ref, be_ref,
           b_ref, w1, v1, w2, v2, w3, v3, out,
           sum_acc, max_acc, cnt_acc):
    i = pl.program_id(0)

    @pl.when(i == 0)
    def _():
      sum_acc[...] = jnp.zeros_like(sum_acc)
      cnt_acc[...] = jnp.zeros_like(cnt_acc)
      max_acc[...] = jnp.full_like(max_acc, -jnp.inf)

    dv = di[...]
    t = jnp.concatenate([al0[...] + al1[...] + qlo[...],
                         ah0[...] + ah1[...] + qhi[...]], axis=1) * dv
    hb = (t + b_ref2[...]) * (g_ref[...] * BN_SCALE) + be_ref[...]
    hb = jnp.maximum(hb, 0.0)
    bv = b_ref[...]
    gids = lax.broadcasted_iota(jnp.int32, (1, ngraphs), 1)
    onehot = (bv == gids).astype(jnp.float32)
    sum_acc[...] += lax.dot_general(
        onehot, hb, (((0,), (0,)), ((), ())),
        preferred_element_type=jnp.float32)
    cnt_acc[...] += lax.dot_general(
        onehot, jnp.ones((r, 1), jnp.float32), (((0,), (0,)), ((), ())),
        preferred_element_type=jnp.float32)

    gmin = jnp.min(bv)
    gmax = jnp.max(bv)

    def gbody(gid, _):
      m = jnp.where(bv == gid, hb, -jnp.inf)
      mg = jnp.max(m, axis=0, keepdims=True)
      max_acc[pl.ds(gid, 1), :] = jnp.maximum(max_acc[pl.ds(gid, 1), :], mg)
      return 0
    lax.fori_loop(gmin, gmax + 1, gbody, 0)

    @pl.when(i == grid - 1)
    def _():
      mean = sum_acc[...] / jnp.maximum(cnt_acc[...], 1.0)
      mx = max_acc[...]
      mx = jnp.where(jnp.isfinite(mx), mx, 0.0)
      z = jnp.concatenate([mean, mx], axis=1)
      z = gelu(jnp.dot(z, w1[...], preferred_element_type=jnp.float32)
               + v1[...])
      z = gelu(jnp.dot(z, w2[...], preferred_element_type=jnp.float32)
               + v2[...])
      out[...] = (jnp.dot(z, w3[...], preferred_element_type=jnp.float32)
                  + v3[...])

  return pl.pallas_call(
      body,
      grid=(grid,),
      in_specs=(
          [pl.BlockSpec((r, d // 2), lambda i: (i, 0))] * 4 +
          [pl.BlockSpec((r, d // 2), lambda i: (i, 0))] * 2 +
          [pl.BlockSpec((r, 1), lambda i: (i, 0)),
           pl.BlockSpec((1, d), lambda i: (0, 0)),
           pl.BlockSpec((1, d), lambda i: (0, 0)),
           pl.BlockSpec((1, d), lambda i: (0, 0)),
           pl.BlockSpec((r, 1), lambda i: (i, 0)),
           pl.BlockSpec((2 * d, d1), lambda i: (0, 0)),
           pl.BlockSpec((1, d1), lambda i: (0, 0)),
           pl.BlockSpec((d1, d2), lambda i: (0, 0)),
           pl.BlockSpec((1, d2), lambda i: (0, 0)),
           pl.BlockSpec((d2, d3), lambda i: (0, 0)),
           pl.BlockSpec((1, d3), lambda i: (0, 0))]),
      out_specs=pl.BlockSpec((ngraphs, d3), lambda i: (0, 0)),
      out_shape=jax.ShapeDtypeStruct((ngraphs, d3), jnp.float32),
      scratch_shapes=[
          pltpu.VMEM((ngraphs, d), jnp.float32),
          pltpu.VMEM((ngraphs, d), jnp.float32),
          pltpu.VMEM((ngraphs, 1), jnp.float32),
      ],
  )(*a_parts, *q_parts, dinv, b.reshape(1, d), g.reshape(1, d),
    be.reshape(1, d), batch.reshape(n, 1), wm1, bm1.reshape(1, d1),
    wm2, bm2.reshape(1, d2), wm3, bm3.reshape(1, d3))


def kernel(x, edge_index, batch, W1, b1, g1, be1, W2, b2, g2, be2,
           W3, b3, g3, be3, Wm1, bm1, Wm2, bm2, Wm3, bm3):
  n, d = x.shape
  e = edge_index.shape[1]
  ngraphs = 64

  src2 = edge_index[0].reshape(e // CHUNK, 1, CHUNK)
  dst2 = edge_index[1].reshape(e // CHUNK, 1, CHUNK)

  deg0, deg1 = _sc_degree(dst2, n)

  dinv, q1 = _tc_prep(x, deg0, deg1, W1)
  a0, a1 = _sc_edge_sum_part(q1, src2, dst2)
  p2 = _tc_post1(a0, a1, q1, dinv, b1, g1, be1)

  a0, a1 = _sc_edge_sum_part(p2, src2, dst2)
  q3_lo, q3_hi = _tc_dense2(a0, a1, p2, dinv, W2, b2, g2, be2, W3)

  al0, al1 = _sc_edge_sum_part(q3_lo, src2, dst2)
  ah0, ah1 = _sc_edge_sum_part(q3_hi, src2, dst2)

  return _tc_pool_mlp((al0, al1, ah0, ah1), (q3_lo, q3_hi), dinv,
                      b3, g3, be3, batch, ngraphs,
                      Wm1, bm1, Wm2, bm2, Wm3, bm3)
